# Initial kernel scaffold; baseline (speedup 1.0000x reference)
import jax
import jax.numpy as jnp
from jax.experimental import pallas as pl


def kernel(x):
    raise NotImplementedError("write your pallas kernel here")


# speedup vs baseline: 10.2715x; 10.2715x over previous
import functools
import math

import jax
import jax.numpy as jnp
from jax import lax
from jax.experimental import pallas as pl
from jax.experimental.pallas import tpu as pltpu
from jax.experimental.pallas import tpu_sc as plsc

_SPARSITY = 0.05
_L = 16

_SH1 = 21
_SH2 = 10
_NB12 = 1 << 11
_NB3 = 1 << 10
_INT_MAX = 2147483647


def _monotone_key(xc):
    s = lax.bitcast_convert_type(xc, jnp.int32)
    m = lax.shift_right_arithmetic(s, 31)
    t = s ^ (m | jnp.int32(-2147483648))
    return lax.bitcast_convert_type(t, jnp.uint32)


def _zero_hist(hist_ref, nbins):
    z = jnp.zeros((_L,), jnp.int32)

    def body(j, carry):
        hist_ref[pl.ds(j * _L, _L)] = z
        return carry

    lax.fori_loop(0, nbins // _L, body, jnp.int32(0))


def _select_digit(hist_ref, nbins, rk):
    iota = lax.iota(jnp.int32, _L)
    nch = nbins // _L

    def body(j, carry):
        acc, found, digit, rko = carry
        base = nbins - (j + 1) * _L
        v = hist_ref[pl.ds(base, _L)]
        rv = lax.rev(v, (0,))
        c = plsc.cumsum(rv) + acc
        hitv = c >= rk
        hit = jnp.any(hitv).astype(jnp.int32)
        pstar = jnp.min(jnp.where(hitv, iota, jnp.int32(_L)))
        cge = jnp.min(jnp.where(hitv, c, jnp.int32(_INT_MAX)))
        hd = jnp.max(jnp.where(iota == pstar, rv, jnp.int32(0)))
        dnew = base + (_L - 1) - pstar
        rknew = rk - (cge - hd)
        take = (hit == 1) & (found == 0)
        digit = jnp.where(take, dnew, digit)
        rko = jnp.where(take, rknew, rko)
        found = found | hit
        acc = acc + jnp.sum(v)
        return acc, found, digit, rko

    init = (jnp.int32(0), jnp.int32(0), jnp.int32(0), rk)
    _, _, digit, rko = lax.fori_loop(0, nch, body, init)
    return digit, rko


@functools.cache
def _build(batch, emb):
    nchunk = emb // _L
    k_active = math.ceil(_SPARSITY * emb)
    info = plsc.get_sparse_core_info()
    nworkers = info.num_cores * info.num_subcores
    rows_per = batch // nworkers
    mesh = plsc.VectorSubcoreMesh(core_axis_name="c", subcore_axis_name="s")

    def hist_pass(kv, hist_ref, shift, dmask, pshift, prefix):
        ones = jnp.ones((_L,), jnp.int32)

        def body(i, carry):
            kc = kv[pl.ds(i * _L, _L)]
            digit = ((kc >> jnp.uint32(shift)) & jnp.uint32(dmask)).astype(
                jnp.int32)
            if prefix is None:
                plsc.addupdate_scatter(hist_ref, [digit], ones)
            else:
                m = (kc >> jnp.uint32(pshift)) == prefix
                plsc.addupdate_scatter(hist_ref, [digit], ones, mask=m)
            return carry

        lax.fori_loop(0, nchunk, body, jnp.int32(0))

    @functools.partial(
        pl.kernel,
        mesh=mesh,
        out_type=jax.ShapeDtypeStruct((batch, emb), jnp.float32),
        compiler_params=pltpu.CompilerParams(needs_layout_passes=False),
        scratch_types=[
            pltpu.VMEM((emb,), jnp.float32),
            pltpu.VMEM((emb,), jnp.uint32),
            pltpu.VMEM((_NB12,), jnp.int32),
        ],
    )
    def kwta(x_hbm, out_hbm, xv, kv, hist):
        cid = lax.axis_index("c")
        sid = lax.axis_index("s")
        wid = sid * info.num_cores + cid

        for rr in range(rows_per):
            r = wid * rows_per + rr
            pltpu.sync_copy(x_hbm.at[r], xv)

            def key_body(i, carry):
                kv[pl.ds(i * _L, _L)] = _monotone_key(xv[pl.ds(i * _L, _L)])
                return carry

            lax.fori_loop(0, nchunk, key_body, jnp.int32(0))

            _zero_hist(hist, _NB12)
            hist_pass(kv, hist, _SH1, _NB12 - 1, 0, None)
            d1, rk2 = _select_digit(hist, _NB12, jnp.int32(k_active))
            p1 = d1.astype(jnp.uint32)

            _zero_hist(hist, _NB12)
            hist_pass(kv, hist, _SH2, _NB12 - 1, _SH1, p1)
            d2, rk3 = _select_digit(hist, _NB12, rk2)
            p2 = (p1 << jnp.uint32(11)) | d2.astype(jnp.uint32)

            _zero_hist(hist, _NB3)
            hist_pass(kv, hist, 0, _NB3 - 1, _SH2, p2)
            d3, _ = _select_digit(hist, _NB3, rk3)
            thresh = (p2 << jnp.uint32(10)) | d3.astype(jnp.uint32)

            def mask_body(i, carry):
                kc = kv[pl.ds(i * _L, _L)]
                xv[pl.ds(i * _L, _L)] = jnp.where(
                    kc >= thresh, jnp.float32(1.0), jnp.float32(0.0))
                return carry

            lax.fori_loop(0, nchunk, mask_body, jnp.int32(0))
            pltpu.sync_copy(xv, out_hbm.at[r])

    return kwta


def kernel(x):
    batch, emb = x.shape
    return _build(batch, emb)(x)


# speedup vs baseline: 11.9180x; 1.1603x over previous
import functools
import math

import jax
import jax.numpy as jnp
from jax import lax
from jax.experimental import pallas as pl
from jax.experimental.pallas import tpu as pltpu
from jax.experimental.pallas import tpu_sc as plsc

_SPARSITY = 0.05
_L = 16

_SH1 = 21
_SH2 = 10
_NB12 = 1 << 11
_NB3 = 1 << 10
_INT_MAX = 2147483647


def _monotone_key(xc):
    s = lax.bitcast_convert_type(xc, jnp.int32)
    m = lax.shift_right_arithmetic(s, 31)
    t = s ^ (m | jnp.int32(-2147483648))
    return lax.bitcast_convert_type(t, jnp.uint32)


def _zero_hist(hist_ref, nbins):
    z = jnp.zeros((_L,), jnp.int32)

    def body(j, carry):
        hist_ref[pl.ds(j * _L, _L)] = z
        return carry

    lax.fori_loop(0, nbins // _L, body, jnp.int32(0), unroll=8)


def _select_digit(hist_ref, nbins, rk):
    iota = lax.iota(jnp.int32, _L)
    nch = nbins // _L

    def body(j, carry):
        acc, found, digit, rko = carry
        base = nbins - (j + 1) * _L
        v = hist_ref[pl.ds(base, _L)]
        rv = lax.rev(v, (0,))
        c = plsc.cumsum(rv) + acc
        hitv = c >= rk
        hit = jnp.any(hitv).astype(jnp.int32)
        pstar = jnp.min(jnp.where(hitv, iota, jnp.int32(_L)))
        cge = jnp.min(jnp.where(hitv, c, jnp.int32(_INT_MAX)))
        hd = jnp.max(jnp.where(iota == pstar, rv, jnp.int32(0)))
        dnew = base + (_L - 1) - pstar
        rknew = rk - (cge - hd)
        take = (hit == 1) & (found == 0)
        digit = jnp.where(take, dnew, digit)
        rko = jnp.where(take, rknew, rko)
        found = found | hit
        acc = acc + jnp.sum(v)
        return acc, found, digit, rko

    init = (jnp.int32(0), jnp.int32(0), jnp.int32(0), rk)
    _, _, digit, rko = lax.fori_loop(0, nch, body, init, unroll=2)
    return digit, rko


@functools.cache
def _build(batch, emb):
    nchunk = emb // _L
    k_active = math.ceil(_SPARSITY * emb)
    info = plsc.get_sparse_core_info()
    nworkers = info.num_cores * info.num_subcores
    rows_per = batch // nworkers
    mesh = plsc.VectorSubcoreMesh(core_axis_name="c", subcore_axis_name="s")

    def hist_pass(kv, hist_ref, shift, dmask, pshift, prefix, xv=None):
        ones = jnp.ones((_L,), jnp.int32)

        def body(i, carry):
            if xv is not None:
                kc = _monotone_key(xv[pl.ds(i * _L, _L)])
                kv[pl.ds(i * _L, _L)] = kc
            else:
                kc = kv[pl.ds(i * _L, _L)]
            digit = ((kc >> jnp.uint32(shift)) & jnp.uint32(dmask)).astype(
                jnp.int32)
            if prefix is None:
                plsc.addupdate_scatter(hist_ref, [digit], ones)
            else:
                m = (kc >> jnp.uint32(pshift)) == prefix
                plsc.addupdate_scatter(hist_ref, [digit], ones, mask=m)
            return carry

        lax.fori_loop(0, nchunk, body, jnp.int32(0), unroll=8)

    @functools.partial(
        pl.kernel,
        mesh=mesh,
        out_type=jax.ShapeDtypeStruct((batch, emb), jnp.float32),
        compiler_params=pltpu.CompilerParams(needs_layout_passes=False),
        scratch_types=[
            pltpu.VMEM((emb,), jnp.float32),
            pltpu.VMEM((emb,), jnp.uint32),
            pltpu.VMEM((_NB12,), jnp.int32),
        ],
    )
    def kwta(x_hbm, out_hbm, xv, kv, hist):
        cid = lax.axis_index("c")
        sid = lax.axis_index("s")
        wid = sid * info.num_cores + cid

        for rr in range(rows_per):
            r = wid * rows_per + rr
            pltpu.sync_copy(x_hbm.at[r], xv)

            _zero_hist(hist, _NB12)
            hist_pass(kv, hist, _SH1, _NB12 - 1, 0, None, xv=xv)
            d1, rk2 = _select_digit(hist, _NB12, jnp.int32(k_active))
            p1 = d1.astype(jnp.uint32)

            _zero_hist(hist, _NB12)
            hist_pass(kv, hist, _SH2, _NB12 - 1, _SH1, p1)
            d2, rk3 = _select_digit(hist, _NB12, rk2)
            p2 = (p1 << jnp.uint32(11)) | d2.astype(jnp.uint32)

            _zero_hist(hist, _NB3)
            hist_pass(kv, hist, 0, _NB3 - 1, _SH2, p2)
            d3, _ = _select_digit(hist, _NB3, rk3)
            thresh = (p2 << jnp.uint32(10)) | d3.astype(jnp.uint32)

            def mask_body(i, carry):
                kc = kv[pl.ds(i * _L, _L)]
                xv[pl.ds(i * _L, _L)] = jnp.where(
                    kc >= thresh, jnp.float32(1.0), jnp.float32(0.0))
                return carry

            lax.fori_loop(0, nchunk, mask_body, jnp.int32(0), unroll=8)
            pltpu.sync_copy(xv, out_hbm.at[r])

    return kwta


def kernel(x):
    batch, emb = x.shape
    return _build(batch, emb)(x)
